# baseline (speedup 1.0000x reference)
import functools

import jax
import jax.numpy as jnp
from jax import lax
from jax.experimental import pallas as pl
from jax.experimental.pallas import tpu as pltpu
from jax.experimental.pallas import tpu_sc as plsc

B, L, D = 4096, 200, 64
DP = 128
NC, NS = 2, 16
NW = NC * NS
BPW = B // NW
LC = 2
NCH = L // LC
RPC = LC * BPW
ECR = LC * D
NLG = BPW // 16


def _make_sc_call():
    mesh = plsc.VectorSubcoreMesh(core_axis_name="c", subcore_axis_name="s")

    @functools.partial(
        pl.kernel,
        out_type=jax.ShapeDtypeStruct((D, B), jnp.float32),
        mesh=mesh,
        compiler_params=pltpu.CompilerParams(use_tc_tiling_on_sc=True,
                                             needs_layout_passes=False),
        scratch_types=[
            pltpu.VMEM((RPC,), jnp.int32),
            pltpu.VMEM((RPC,), jnp.int32),
            pltpu.VMEM((RPC, DP), jnp.float32),
            pltpu.VMEM((RPC, DP), jnp.float32),
            pltpu.VMEM((ECR, BPW), jnp.float32),
            pltpu.VMEM((ECR, BPW), jnp.float32),
            pltpu.VMEM((D, BPW), jnp.float32),
            pltpu.SemaphoreType.DMA,
            pltpu.SemaphoreType.DMA,
            pltpu.SemaphoreType.DMA,
            pltpu.SemaphoreType.DMA,
        ],
    )
    def sc_kernel(rid_hbm, e_hbm, tab_hbm, out_hbm,
                  i0_v, i1_v, w0_v, w1_v, e0_v, e1_v, acc_v,
                  si0, si1, sd0, sd1):
        wid = lax.axis_index("s") * NC + lax.axis_index("c")
        wb0 = wid * BPW

        def idx_cp(c, idx_v, sem):
            l0 = c * LC
            return [
                pltpu.make_async_copy(
                    rid_hbm.at[pl.ds(l0 * B + wb0, BPW)],
                    idx_v.at[pl.ds(0, BPW)], sem),
                pltpu.make_async_copy(
                    rid_hbm.at[pl.ds((l0 + 1) * B + wb0, BPW)],
                    idx_v.at[pl.ds(BPW, BPW)], sem),
            ]

        def dat_cp(c, idx_v, w_v, e_v, sem):
            return [
                pltpu.make_async_copy(
                    tab_hbm.at[idx_v.at[pl.ds(0, BPW)]],
                    w_v.at[pl.ds(0, BPW)], sem),
                pltpu.make_async_copy(
                    tab_hbm.at[idx_v.at[pl.ds(BPW, BPW)]],
                    w_v.at[pl.ds(BPW, BPW)], sem),
                pltpu.make_async_copy(
                    e_hbm.at[pl.ds(c * ECR, ECR), pl.ds(wb0, BPW)],
                    e_v, sem),
            ]

        def start(descs):
            for d_ in descs:
                d_.start()

        def wait(descs):
            for d_ in descs:
                d_.wait()

        iota = lax.iota(jnp.int32, 16)
        zvec = jnp.zeros((16,), jnp.float32)
        one = jnp.full((16,), 1, jnp.int32)
        eps = jnp.full((16,), 1e-24, jnp.float32)

        def zero_acc(d_, carry):
            for g in range(NLG):
                acc_v[d_, pl.ds(g * 16, 16)] = zvec
            return carry

        lax.fori_loop(0, D, zero_acc, 0)

        def compute(w_v, e_v):
            def lg_body(lg, carry):
                col0 = lg * 16
                for l_rel in range(LC):
                    rows = (l_rel * BPW + col0 + iota) * jnp.int32(1)
                    cols = jnp.zeros((16,), jnp.int32)
                    dot = zvec
                    nsq = zvec
                    for d_ in range(D):
                        w_vec = plsc.load_gather(w_v, [rows, cols])
                        e_vec = e_v[l_rel * D + d_, pl.ds(col0, 16)]
                        dot = dot + e_vec * w_vec
                        nsq = nsq + w_vec * w_vec
                        plsc.addupdate(acc_v.at[d_, pl.ds(col0, 16)], e_vec)
                        cols = cols + one
                    nc = -(dot / jnp.maximum(nsq, eps))
                    cols = jnp.zeros((16,), jnp.int32)
                    for d_ in range(D):
                        w_vec = plsc.load_gather(w_v, [rows, cols])
                        plsc.addupdate(acc_v.at[d_, pl.ds(col0, 16)],
                                       nc * w_vec)
                        cols = cols + one
                return carry

            lax.fori_loop(0, NLG, lg_body, 0)

        start(idx_cp(0, i0_v, si0))
        start(idx_cp(1, i1_v, si1))
        wait(idx_cp(0, i0_v, si0))
        start(dat_cp(0, i0_v, w0_v, e0_v, sd0))

        def pair(j, carry):
            c0 = 2 * j
            wait(idx_cp(c0 + 1, i1_v, si1))
            start(dat_cp(c0 + 1, i1_v, w1_v, e1_v, sd1))
            wait(dat_cp(c0, i0_v, w0_v, e0_v, sd0))

            @pl.when(j < NCH // 2 - 1)
            def _():
                start(idx_cp(c0 + 2, i0_v, si0))

            compute(w0_v, e0_v)

            @pl.when(j < NCH // 2 - 1)
            def _():
                wait(idx_cp(c0 + 2, i0_v, si0))
                start(dat_cp(c0 + 2, i0_v, w0_v, e0_v, sd0))

            wait(dat_cp(c0 + 1, i1_v, w1_v, e1_v, sd1))

            @pl.when(j < NCH // 2 - 1)
            def _():
                start(idx_cp(c0 + 3, i1_v, si1))

            compute(w1_v, e1_v)
            return carry

        lax.fori_loop(0, NCH // 2, pair, 0)

        scale = jnp.float32(L)

        def fin(d_, carry):
            for g in range(NLG):
                acc_v[d_, pl.ds(g * 16, 16)] = (
                    acc_v[d_, pl.ds(g * 16, 16)] / scale)
            return carry

        lax.fori_loop(0, D, fin, 0)
        pltpu.sync_copy(acc_v, out_hbm.at[pl.ds(0, D), pl.ds(wb0, BPW)])

    return sc_kernel


_sc_call = _make_sc_call()


def kernel(batch_nei_rid, batch_nei_e_emb, w_r_table):
    rid_t = jnp.transpose(batch_nei_rid, (1, 0)).reshape(L * B)
    e_t = jnp.transpose(batch_nei_e_emb, (1, 2, 0)).reshape(L * D, B)
    tab_pad = jnp.pad(w_r_table, ((0, 0), (0, DP - D)))
    out_t = _sc_call(rid_t, e_t, tab_pad)
    return jnp.transpose(out_t, (1, 0))


# speedup vs baseline: 2.8119x; 2.8119x over previous
import functools

import jax
import jax.numpy as jnp
from jax import lax
from jax.experimental import pallas as pl
from jax.experimental.pallas import tpu as pltpu
from jax.experimental.pallas import tpu_sc as plsc

B, L, D = 4096, 200, 64
DP = 128
NC, NS = 2, 16
NW = NC * NS
BPW = B // NW
HB = BPW // 2
CH0, CH1 = 104, 96


def _make_sc_call():
    mesh = plsc.VectorSubcoreMesh(core_axis_name="c", subcore_axis_name="s")

    @functools.partial(
        pl.kernel,
        out_type=jax.ShapeDtypeStruct((B * D,), jnp.float32),
        mesh=mesh,
        compiler_params=pltpu.CompilerParams(use_tc_tiling_on_sc=True),
        scratch_types=[
            pltpu.VMEM((HB * L,), jnp.int32),
            pltpu.VMEM((L, DP), jnp.float32),
            pltpu.VMEM((L, DP), jnp.float32),
            pltpu.VMEM((L * D,), jnp.float32),
            pltpu.VMEM((L * D,), jnp.float32),
            pltpu.VMEM((BPW * D,), jnp.float32),
            pltpu.SemaphoreType.DMA,
            pltpu.SemaphoreType.DMA,
        ],
    )
    def sc_kernel(rid_hbm, e_hbm, tab_hbm, out_hbm,
                  idx_v, w0_v, w1_v, e0_v, e1_v, out_v, sem0, sem1):
        wid = lax.axis_index("s") * NC + lax.axis_index("c")
        base = wid * BPW

        def load_idx(half):
            pltpu.sync_copy(
                rid_hbm.at[pl.ds((base + half * HB) * L, HB * L)], idx_v)

        def fetch(bi, bg, w_v, e_v, sem):
            pltpu.async_copy(tab_hbm.at[idx_v.at[pl.ds(bi * L, CH0)]],
                             w_v.at[pl.ds(0, CH0)], sem)
            pltpu.async_copy(tab_hbm.at[idx_v.at[pl.ds(bi * L + CH0, CH1)]],
                             w_v.at[pl.ds(CH0, CH1)], sem)
            pltpu.async_copy(e_hbm.at[pl.ds((base + bg) * (L * D), L * D)],
                             e_v, sem)

        def wait(bi, bg, w_v, e_v, sem):
            pltpu.make_async_copy(tab_hbm.at[idx_v.at[pl.ds(bi * L, CH0)]],
                                  w_v.at[pl.ds(0, CH0)], sem).wait()
            pltpu.make_async_copy(tab_hbm.at[idx_v.at[pl.ds(bi * L + CH0, CH1)]],
                                  w_v.at[pl.ds(CH0, CH1)], sem).wait()
            pltpu.make_async_copy(
                e_hbm.at[pl.ds((base + bg) * (L * D), L * D)], e_v, sem).wait()

        lanes = lax.iota(jnp.int32, 16)
        perms = [(lanes ^ k)[:, None] for k in (8, 4, 2, 1)]
        _dnums = lax.GatherDimensionNumbers(
            offset_dims=(), collapsed_slice_dims=(0,), start_index_map=(0,))

        def hsum(x):
            for p in perms:
                x = x + lax.gather(
                    x, p, _dnums, (1,),
                    mode=lax.GatherScatterMode.PROMISE_IN_BOUNDS)
            return x

        UNROLL = 4
        assert L % UNROLL == 0

        def compute(bl, w_v, e_v):
            def body(i, accs):
                a0, a1, a2, a3 = accs
                lb = i * UNROLL
                for u in range(UNROLL):
                    l = lb + u
                    w0 = w_v[l, pl.ds(0, 16)]
                    w1 = w_v[l, pl.ds(16, 16)]
                    w2 = w_v[l, pl.ds(32, 16)]
                    w3 = w_v[l, pl.ds(48, 16)]
                    e0 = e_v[pl.ds(l * D, 16)]
                    e1 = e_v[pl.ds(l * D + 16, 16)]
                    e2 = e_v[pl.ds(l * D + 32, 16)]
                    e3 = e_v[pl.ds(l * D + 48, 16)]
                    nsq = w0 * w0 + w1 * w1 + w2 * w2 + w3 * w3
                    dot = e0 * w0 + e1 * w1 + e2 * w2 + e3 * w3
                    ns = hsum(nsq)
                    dt = hsum(dot)
                    c = dt / jnp.maximum(ns, jnp.float32(1e-24))
                    a0 = a0 + (e0 - c * w0)
                    a1 = a1 + (e1 - c * w1)
                    a2 = a2 + (e2 - c * w2)
                    a3 = a3 + (e3 - c * w3)
                return (a0, a1, a2, a3)

            z = jnp.zeros((16,), jnp.float32)
            a0, a1, a2, a3 = lax.fori_loop(0, L // UNROLL, body, (z, z, z, z))
            scale = jnp.float32(L)
            ob = bl * D
            out_v[pl.ds(ob, 16)] = a0 / scale
            out_v[pl.ds(ob + 16, 16)] = a1 / scale
            out_v[pl.ds(ob + 32, 16)] = a2 / scale
            out_v[pl.ds(ob + 48, 16)] = a3 / scale

        def half_loop(half):
            hb0 = half * HB
            fetch(0, hb0, w0_v, e0_v, sem0)

            def pair(j, carry):
                bi = 2 * j
                bg = hb0 + bi
                fetch(bi + 1, bg + 1, w1_v, e1_v, sem1)
                wait(bi, bg, w0_v, e0_v, sem0)
                compute(bg, w0_v, e0_v)

                @pl.when(j < HB // 2 - 1)
                def _():
                    fetch(bi + 2, bg + 2, w0_v, e0_v, sem0)

                wait(bi + 1, bg + 1, w1_v, e1_v, sem1)
                compute(bg + 1, w1_v, e1_v)
                return carry

            lax.fori_loop(0, HB // 2, pair, 0)

        load_idx(0)
        half_loop(0)
        load_idx(1)
        half_loop(1)
        pltpu.sync_copy(out_v, out_hbm.at[pl.ds(base * D, BPW * D)])

    return sc_kernel


_sc_call = _make_sc_call()


def kernel(batch_nei_rid, batch_nei_e_emb, w_r_table):
    rid_flat = batch_nei_rid.reshape(B * L)
    e_flat = batch_nei_e_emb.reshape(B * L * D)
    tab_pad = jnp.pad(w_r_table, ((0, 0), (0, DP - D)))
    out = _sc_call(rid_flat, e_flat, tab_pad)
    return out.reshape(B, D)


# speedup vs baseline: 3.3426x; 1.1887x over previous
import functools

import jax
import jax.numpy as jnp
from jax import lax
from jax.experimental import pallas as pl
from jax.experimental.pallas import tpu as pltpu
from jax.experimental.pallas import tpu_sc as plsc

B, L, D = 4096, 200, 64
DP = 128
NC, NS = 2, 16
NW = NC * NS
BPW = B // NW
HB = BPW // 2
CH0, CH1 = 104, 96


def _make_sc_call():
    mesh = plsc.VectorSubcoreMesh(core_axis_name="c", subcore_axis_name="s")

    @functools.partial(
        pl.kernel,
        out_type=jax.ShapeDtypeStruct((B * D,), jnp.float32),
        mesh=mesh,
        compiler_params=pltpu.CompilerParams(use_tc_tiling_on_sc=True),
        scratch_types=[
            pltpu.VMEM((HB * L,), jnp.int32),
            pltpu.VMEM((L, DP), jnp.float32),
            pltpu.VMEM((L, DP), jnp.float32),
            pltpu.VMEM((L, D), jnp.float32),
            pltpu.VMEM((L, D), jnp.float32),
            pltpu.VMEM((BPW * D,), jnp.float32),
            pltpu.SemaphoreType.DMA,
            pltpu.SemaphoreType.DMA,
        ],
    )
    def sc_kernel(rid_hbm, e_hbm, tab_hbm, out_hbm,
                  idx_v, w0_v, w1_v, e0_v, e1_v, out_v, sem0, sem1):
        wid = lax.axis_index("s") * NC + lax.axis_index("c")
        base = wid * BPW

        def load_idx(half):
            pltpu.sync_copy(
                rid_hbm.at[pl.ds((base + half * HB) * L, HB * L)], idx_v)

        def fetch(bi, bg, w_v, e_v, sem):
            pltpu.async_copy(tab_hbm.at[idx_v.at[pl.ds(bi * L, CH0)]],
                             w_v.at[pl.ds(0, CH0)], sem)
            pltpu.async_copy(tab_hbm.at[idx_v.at[pl.ds(bi * L + CH0, CH1)]],
                             w_v.at[pl.ds(CH0, CH1)], sem)
            pltpu.async_copy(e_hbm.at[base + bg], e_v, sem)

        def wait(bi, bg, w_v, e_v, sem):
            pltpu.make_async_copy(tab_hbm.at[idx_v.at[pl.ds(bi * L, CH0)]],
                                  w_v.at[pl.ds(0, CH0)], sem).wait()
            pltpu.make_async_copy(tab_hbm.at[idx_v.at[pl.ds(bi * L + CH0, CH1)]],
                                  w_v.at[pl.ds(CH0, CH1)], sem).wait()
            pltpu.make_async_copy(e_hbm.at[base + bg], e_v, sem).wait()

        lanes = lax.iota(jnp.int32, 16)
        perms = [(lanes ^ k)[:, None] for k in (8, 4, 2, 1)]
        _dnums = lax.GatherDimensionNumbers(
            offset_dims=(), collapsed_slice_dims=(0,), start_index_map=(0,))

        def hsum(x):
            for p in perms:
                x = x + lax.gather(
                    x, p, _dnums, (1,),
                    mode=lax.GatherScatterMode.PROMISE_IN_BOUNDS)
            return x

        UNROLL = 4
        assert L % UNROLL == 0

        def compute(bl, w_v, e_v):
            def body(i, accs):
                a0, a1, a2, a3 = accs
                lb = i * UNROLL
                for u in range(UNROLL):
                    l = lb + u
                    w0 = w_v[l, pl.ds(0, 16)]
                    w1 = w_v[l, pl.ds(16, 16)]
                    w2 = w_v[l, pl.ds(32, 16)]
                    w3 = w_v[l, pl.ds(48, 16)]
                    e0 = e_v[l, pl.ds(0, 16)]
                    e1 = e_v[l, pl.ds(16, 16)]
                    e2 = e_v[l, pl.ds(32, 16)]
                    e3 = e_v[l, pl.ds(48, 16)]
                    nsq = w0 * w0 + w1 * w1 + w2 * w2 + w3 * w3
                    dot = e0 * w0 + e1 * w1 + e2 * w2 + e3 * w3
                    ns = hsum(nsq)
                    dt = hsum(dot)
                    c = dt / jnp.maximum(ns, jnp.float32(1e-24))
                    a0 = a0 + (e0 - c * w0)
                    a1 = a1 + (e1 - c * w1)
                    a2 = a2 + (e2 - c * w2)
                    a3 = a3 + (e3 - c * w3)
                return (a0, a1, a2, a3)

            z = jnp.zeros((16,), jnp.float32)
            a0, a1, a2, a3 = lax.fori_loop(0, L // UNROLL, body, (z, z, z, z))
            scale = jnp.float32(L)
            ob = bl * D
            out_v[pl.ds(ob, 16)] = a0 / scale
            out_v[pl.ds(ob + 16, 16)] = a1 / scale
            out_v[pl.ds(ob + 32, 16)] = a2 / scale
            out_v[pl.ds(ob + 48, 16)] = a3 / scale

        def half_loop(half):
            hb0 = half * HB
            fetch(0, hb0, w0_v, e0_v, sem0)

            def pair(j, carry):
                bi = 2 * j
                bg = hb0 + bi
                fetch(bi + 1, bg + 1, w1_v, e1_v, sem1)
                wait(bi, bg, w0_v, e0_v, sem0)
                compute(bg, w0_v, e0_v)

                @pl.when(j < HB // 2 - 1)
                def _():
                    fetch(bi + 2, bg + 2, w0_v, e0_v, sem0)

                wait(bi + 1, bg + 1, w1_v, e1_v, sem1)
                compute(bg + 1, w1_v, e1_v)
                return carry

            lax.fori_loop(0, HB // 2, pair, 0)

        load_idx(0)
        half_loop(0)
        load_idx(1)
        half_loop(1)
        pltpu.sync_copy(out_v, out_hbm.at[pl.ds(base * D, BPW * D)])

    return sc_kernel


_sc_call = _make_sc_call()


def kernel(batch_nei_rid, batch_nei_e_emb, w_r_table):
    rid_flat = batch_nei_rid.reshape(B * L)
    tab_pad = jnp.pad(w_r_table, ((0, 0), (0, DP - D)))
    out = _sc_call(rid_flat, batch_nei_e_emb, tab_pad)
    return out.reshape(B, D)


# speedup vs baseline: 3.5831x; 1.0719x over previous
import functools

import jax
import jax.numpy as jnp
from jax import lax
from jax.experimental import pallas as pl
from jax.experimental.pallas import tpu as pltpu
from jax.experimental.pallas import tpu_sc as plsc

B, L, D = 4096, 200, 64
DP = 128
NC, NS = 2, 16
NW = NC * NS
BPW = B // NW
HB = BPW // 2
CH0, CH1 = 104, 96


def _make_sc_call():
    mesh = plsc.VectorSubcoreMesh(core_axis_name="c", subcore_axis_name="s")

    @functools.partial(
        pl.kernel,
        out_type=jax.ShapeDtypeStruct((B * D,), jnp.float32),
        mesh=mesh,
        compiler_params=pltpu.CompilerParams(use_tc_tiling_on_sc=True),
        scratch_types=[
            pltpu.VMEM((HB * L,), jnp.int32),
            pltpu.VMEM((L, DP), jnp.float32),
            pltpu.VMEM((L, DP), jnp.float32),
            pltpu.VMEM((L, D), jnp.float32),
            pltpu.VMEM((L, D), jnp.float32),
            pltpu.VMEM((BPW * D,), jnp.float32),
            pltpu.SemaphoreType.DMA,
            pltpu.SemaphoreType.DMA,
        ],
    )
    def sc_kernel(rid_hbm, e_hbm, tab_hbm, out_hbm,
                  idx_v, w0_v, w1_v, e0_v, e1_v, out_v, sem0, sem1):
        wid = lax.axis_index("s") * NC + lax.axis_index("c")
        base = wid * BPW

        def load_idx(half):
            pltpu.sync_copy(
                rid_hbm.at[pl.ds((base + half * HB) * L, HB * L)], idx_v)

        def fetch(bi, bg, w_v, e_v, sem):
            pltpu.async_copy(tab_hbm.at[idx_v.at[pl.ds(bi * L, CH0)]],
                             w_v.at[pl.ds(0, CH0)], sem)
            pltpu.async_copy(tab_hbm.at[idx_v.at[pl.ds(bi * L + CH0, CH1)]],
                             w_v.at[pl.ds(CH0, CH1)], sem)
            pltpu.async_copy(e_hbm.at[base + bg], e_v, sem)

        def wait(bi, bg, w_v, e_v, sem):
            pltpu.make_async_copy(tab_hbm.at[idx_v.at[pl.ds(bi * L, CH0)]],
                                  w_v.at[pl.ds(0, CH0)], sem).wait()
            pltpu.make_async_copy(tab_hbm.at[idx_v.at[pl.ds(bi * L + CH0, CH1)]],
                                  w_v.at[pl.ds(CH0, CH1)], sem).wait()
            pltpu.make_async_copy(e_hbm.at[base + bg], e_v, sem).wait()

        lanes = lax.iota(jnp.int32, 16)
        perms = [(lanes ^ k)[:, None] for k in (8, 4, 2, 1)]
        _dnums = lax.GatherDimensionNumbers(
            offset_dims=(), collapsed_slice_dims=(0,), start_index_map=(0,))

        def hsum(x):
            for p in perms:
                x = x + lax.gather(
                    x, p, _dnums, (1,),
                    mode=lax.GatherScatterMode.PROMISE_IN_BOUNDS)
            return x

        UNROLL = 4
        assert L % UNROLL == 0

        def compute(bl, w_v, e_v):
            def body(i, accs):
                a0, a1, a2, a3 = accs
                lb = i * UNROLL
                for u in range(UNROLL):
                    l = lb + u
                    w0 = w_v[l, pl.ds(0, 16)]
                    w1 = w_v[l, pl.ds(16, 16)]
                    w2 = w_v[l, pl.ds(32, 16)]
                    w3 = w_v[l, pl.ds(48, 16)]
                    e0 = e_v[l, pl.ds(0, 16)]
                    e1 = e_v[l, pl.ds(16, 16)]
                    e2 = e_v[l, pl.ds(32, 16)]
                    e3 = e_v[l, pl.ds(48, 16)]
                    nsq = w0 * w0 + w1 * w1 + w2 * w2 + w3 * w3
                    dot = e0 * w0 + e1 * w1 + e2 * w2 + e3 * w3
                    ns = hsum(nsq)
                    dt = hsum(dot)
                    c = dt / jnp.maximum(ns, jnp.float32(1e-24))
                    a0 = a0 + (e0 - c * w0)
                    a1 = a1 + (e1 - c * w1)
                    a2 = a2 + (e2 - c * w2)
                    a3 = a3 + (e3 - c * w3)
                return (a0, a1, a2, a3)

            z = jnp.zeros((16,), jnp.float32)
            a0, a1, a2, a3 = lax.fori_loop(0, L // UNROLL, body, (z, z, z, z))
            scale = jnp.float32(L)
            ob = bl * D
            out_v[pl.ds(ob, 16)] = a0 / scale
            out_v[pl.ds(ob + 16, 16)] = a1 / scale
            out_v[pl.ds(ob + 32, 16)] = a2 / scale
            out_v[pl.ds(ob + 48, 16)] = a3 / scale

        def half_loop(half):
            hb0 = half * HB
            fetch(0, hb0, w0_v, e0_v, sem0)

            def pair(j, carry):
                bi = 2 * j
                bg = hb0 + bi
                fetch(bi + 1, bg + 1, w1_v, e1_v, sem1)
                wait(bi, bg, w0_v, e0_v, sem0)
                compute(bg, w0_v, e0_v)

                @pl.when(j < HB // 2 - 1)
                def _():
                    fetch(bi + 2, bg + 2, w0_v, e0_v, sem0)

                wait(bi + 1, bg + 1, w1_v, e1_v, sem1)
                compute(bg + 1, w1_v, e1_v)
                return carry

            lax.fori_loop(0, HB // 2, pair, 0)

        load_idx(0)
        half_loop(0)
        load_idx(1)
        half_loop(1)
        pltpu.sync_copy(out_v, out_hbm.at[pl.ds(base * D, BPW * D)])

    return sc_kernel


_sc_call = _make_sc_call()

ROWS = 200001
RPAD = 200064
TCH = RPAD // 128
KPW = -(-TCH // NW)


def _make_tab_convert():
    mesh = plsc.VectorSubcoreMesh(core_axis_name="c", subcore_axis_name="s")

    @functools.partial(
        pl.kernel,
        out_type=jax.ShapeDtypeStruct((RPAD, DP), jnp.float32),
        mesh=mesh,
        compiler_params=pltpu.CompilerParams(use_tc_tiling_on_sc=True,
                                             needs_layout_passes=False),
        scratch_types=[
            pltpu.VMEM((D, 128), jnp.float32),
            pltpu.VMEM((D, 128), jnp.float32),
            pltpu.VMEM((128, DP), jnp.float32),
            pltpu.VMEM((128, DP), jnp.float32),
            pltpu.SemaphoreType.DMA,
            pltpu.SemaphoreType.DMA,
            pltpu.SemaphoreType.DMA,
            pltpu.SemaphoreType.DMA,
        ],
    )
    def tab_kernel(tab_t_hbm, out_hbm, s0_v, s1_v, o0_v, o1_v,
                   si0, si1, so0, so1):
        wid = lax.axis_index("s") * NC + lax.axis_index("c")
        iota = lax.iota(jnp.int32, 16)

        def cid_of(k):
            return jnp.minimum(wid * KPW + k, TCH - 1)

        def in_cp(k, s_v, sem):
            return pltpu.make_async_copy(
                tab_t_hbm.at[pl.ds(0, D), pl.ds(cid_of(k) * 128, 128)],
                s_v, sem)

        def out_cp(k, o_v, sem):
            return pltpu.make_async_copy(
                o_v, out_hbm.at[pl.ds(cid_of(k) * 128, 128)], sem)

        def transpose(s_v, o_v):
            def row(r, carry):
                cols = jnp.broadcast_to(r, (16,)).astype(jnp.int32)
                for dg in range(4):
                    o_v[r, pl.ds(dg * 16, 16)] = plsc.load_gather(
                        s_v, [dg * 16 + iota, cols])
                return carry

            lax.fori_loop(0, 128, row, 0)

        in_cp(0, s0_v, si0).start()
        in_cp(1, s1_v, si1).start()

        def pair(j, carry):
            k0 = 2 * j
            in_cp(k0, s0_v, si0).wait()

            @pl.when(j > 0)
            def _():
                out_cp(k0 - 2, o0_v, so0).wait()

            transpose(s0_v, o0_v)
            out_cp(k0, o0_v, so0).start()

            @pl.when(k0 + 2 < KPW)
            def _():
                in_cp(k0 + 2, s0_v, si0).start()

            in_cp(k0 + 1, s1_v, si1).wait()

            @pl.when(j > 0)
            def _():
                out_cp(k0 - 1, o1_v, so1).wait()

            transpose(s1_v, o1_v)
            out_cp(k0 + 1, o1_v, so1).start()

            @pl.when(k0 + 3 < KPW)
            def _():
                in_cp(k0 + 3, s1_v, si1).start()

            return carry

        lax.fori_loop(0, KPW // 2, pair, 0)
        in_cp(KPW - 1, s0_v, si0).wait()
        out_cp(KPW - 3, o0_v, so0).wait()
        transpose(s0_v, o0_v)
        out_cp(KPW - 1, o0_v, so0).start()
        out_cp(KPW - 2, o1_v, so1).wait()
        out_cp(KPW - 1, o0_v, so0).wait()

    return tab_kernel


_tab_convert = _make_tab_convert()


def kernel(batch_nei_rid, batch_nei_e_emb, w_r_table):
    rid_flat = batch_nei_rid.reshape(B * L)
    tab_pad = _tab_convert(jnp.transpose(w_r_table, (1, 0)))
    out = _sc_call(rid_flat, batch_nei_e_emb, tab_pad)
    return out.reshape(B, D)
